# baseline (speedup 1.0000x reference)
import jax
import jax.numpy as jnp
from jax import lax
from jax.experimental import pallas as pl
from jax.experimental.pallas import tpu as pltpu
from jax.experimental.pallas import tpu_sc as plsc

_NC = 2
_NS = 16
_NW = _NC * _NS
_C = 4096
_D = 12


def _sc_body(idx_hbm, tab_hbm, out_hbm, ids_v, tab_v, buf_v):
    per_w = idx_hbm.shape[0] // _NW
    wid = lax.axis_index("s") * _NC + lax.axis_index("c")
    base = wid * per_w
    lane = lax.broadcasted_iota(jnp.int32, (16,), 0)
    lane12 = lane * _D

    pltpu.sync_copy(tab_hbm, tab_v)

    @pl.loop(0, per_w // _C)
    def chunk_body(c):
        n0 = base + c * _C
        pltpu.sync_copy(idx_hbm.at[pl.ds(n0, _C)], ids_v)

        @pl.loop(0, _C // 16, unroll=8)
        def vec_body(t):
            iv = ids_v[pl.ds(t * 16, 16)]
            addr0 = iv * _D
            pos0 = lane12 + t * (16 * _D)
            for r in range(_D):
                val = plsc.load_gather(tab_v, [addr0 + r])
                plsc.store_scatter(buf_v, [pos0 + r], val)

        pltpu.sync_copy(buf_v, out_hbm.at[pl.ds(n0 * _D, _C * _D)])


def _sc_lookup(idx_flat, tab_flat):
    n_total = idx_flat.shape[0]
    mesh = plsc.VectorSubcoreMesh(core_axis_name="c", subcore_axis_name="s")
    return pl.kernel(
        _sc_body,
        out_type=jax.ShapeDtypeStruct((n_total * _D,), jnp.float32),
        mesh=mesh,
        compiler_params=pltpu.CompilerParams(needs_layout_passes=False),
        scratch_types=[
            pltpu.VMEM((_C,), jnp.int32),
            pltpu.VMEM((128,), jnp.float32),
            pltpu.VMEM((_C * _D,), jnp.float32),
        ],
    )(idx_flat, tab_flat)


def kernel(inputs, table):
    n_rows, n_cols = inputs.shape
    idx_flat = inputs.reshape(-1)
    tab_flat = jnp.pad(table.reshape(-1), (0, 128 - table.size))
    out_flat = _sc_lookup(idx_flat, tab_flat)
    return out_flat.reshape(n_rows, n_cols, table.shape[1])


# speedup vs baseline: 1.3663x; 1.3663x over previous
import jax
import jax.numpy as jnp
from jax import lax
from jax.experimental import pallas as pl
from jax.experimental.pallas import tpu as pltpu
from jax.experimental.pallas import tpu_sc as plsc

_NC = 2
_NS = 16
_NW = _NC * _NS
_C = 800
_D = 12


def _sc_body(idx_hbm, tab_hbm, out_hbm, ids_v, tab_v, buf_v):
    per_w = idx_hbm.shape[0] // _NW
    wid = lax.axis_index("s") * _NC + lax.axis_index("c")
    base = wid * per_w
    lane = lax.broadcasted_iota(jnp.int32, (16,), 0)
    lane12 = lane * _D

    pltpu.sync_copy(tab_hbm, tab_v)

    @pl.loop(0, per_w // _C)
    def chunk_body(c):
        n0 = base + c * _C
        pltpu.sync_copy(idx_hbm.at[pl.ds(n0, _C)], ids_v)

        @pl.loop(0, _C // 16, unroll=8)
        def vec_body(t):
            iv = ids_v[pl.ds(t * 16, 16)]
            addr0 = iv * _D
            row = t * 16 + lane
            for r in range(_D):
                val = plsc.load_gather(tab_v, [addr0 + r])
                plsc.store_scatter(buf_v, [row, lane * 0 + r], val)

        pltpu.sync_copy(buf_v, out_hbm.at[pl.ds(n0, _C)])


def _sc_lookup(idx_flat, tab_flat):
    n_total = idx_flat.shape[0]
    mesh = plsc.VectorSubcoreMesh(core_axis_name="c", subcore_axis_name="s")
    return pl.kernel(
        _sc_body,
        out_type=jax.ShapeDtypeStruct((n_total, _D), jnp.float32),
        mesh=mesh,
        compiler_params=pltpu.CompilerParams(needs_layout_passes=False),
        scratch_types=[
            pltpu.VMEM((_C,), jnp.int32),
            pltpu.VMEM((128,), jnp.float32),
            pltpu.VMEM((_C, _D), jnp.float32),
        ],
    )(idx_flat, tab_flat)


def kernel(inputs, table):
    n_rows, n_cols = inputs.shape
    idx_flat = inputs.reshape(-1)
    tab_flat = jnp.pad(table.reshape(-1), (0, 128 - table.size))
    out2 = _sc_lookup(idx_flat, tab_flat)
    return out2.reshape(n_rows, n_cols, table.shape[1])
